# Initial kernel scaffold; baseline (speedup 1.0000x reference)
import jax
import jax.numpy as jnp
from jax.experimental import pallas as pl


def kernel(z_, edge_index, edge_attr, W1, b1, W2, b2, linW, linb):
    raise NotImplementedError("write your pallas kernel here")


# speedup vs baseline: 26.9660x; 26.9660x over previous
import jax
import jax.numpy as jnp
from jax.experimental import pallas as pl

_N = 4096


def _iota_body(r_ref, c_ref):
    rows = r_ref.shape[0]
    base = pl.program_id(0) * rows
    r_ref[...] = jax.lax.broadcasted_iota(jnp.int32, r_ref.shape, 0) + base
    c_ref[...] = jax.lax.broadcasted_iota(jnp.int32, c_ref.shape, 1)


def _edge_iota():
    rows_per = 512
    r, c = pl.pallas_call(
        _iota_body,
        grid=(_N // rows_per,),
        out_specs=[pl.BlockSpec((rows_per, _N), lambda i: (i, 0))] * 2,
        out_shape=[jax.ShapeDtypeStruct((_N, _N), jnp.int32)] * 2,
    )()
    return jnp.stack([r.reshape(-1), c.reshape(-1)], axis=0)


def _gcn(x, src, dst, ew, W, b):
    n = x.shape[0]
    loop = jnp.arange(n, dtype=src.dtype)
    s = jnp.concatenate([src, loop])
    t = jnp.concatenate([dst, loop])
    w = jnp.concatenate([ew, jnp.ones((n,), dtype=x.dtype)])
    deg = jnp.zeros((n,), dtype=x.dtype).at[t].add(w)
    dinv = jax.lax.rsqrt(deg)
    norm = dinv[s] * w * dinv[t]
    h = x @ W
    msg = h[s] * norm[:, None]
    out = jnp.zeros((n, h.shape[1]), dtype=x.dtype).at[t].add(msg)
    return out + b


def kernel(z_, edge_index, edge_attr, W1, b1, W2, b2, linW, linb):
    new_edge_index = _edge_iota()
    src, dst = edge_index[0], edge_index[1]
    h = jax.nn.relu(_gcn(z_, src, dst, edge_attr, W1, b1))
    h = jax.nn.relu(_gcn(h, src, dst, edge_attr, W2, b2))
    out = h @ linW + linb
    return (out, new_edge_index)


# speedup vs baseline: 48.8146x; 1.8102x over previous
import functools

import jax
import jax.numpy as jnp
from jax import lax
from jax.experimental import pallas as pl
from jax.experimental.pallas import tpu as pltpu
from jax.experimental.pallas import tpu_sc as plsc

_N = 4096
_E = 65536
_NC = 2
_NS = 16
_NW = _NC * _NS
_EPW = _E // _NW
_CHUNK = 128
_NCHUNK = _EPW // _CHUNK

_MESH = dict(core_axis_name="c", subcore_axis_name="s")
_SC_PARAMS = pltpu.CompilerParams(
    needs_layout_passes=False, use_tc_tiling_on_sc=False)
_DOT = dict(preferred_element_type=jnp.float32, precision=lax.Precision.HIGHEST)


def _deg_body(dst_hbm, ew_hbm, out_hbm, dst_v, ew_v, deg_v):
    c = lax.axis_index("c")
    s = lax.axis_index("s")
    wid = s * _NC + c
    pltpu.sync_copy(dst_hbm.at[wid], dst_v)
    pltpu.sync_copy(ew_hbm.at[wid], ew_v)

    def zero(i, carry):
        deg_v[pl.ds(pl.multiple_of(i * 16, 16), 16)] = jnp.zeros((16,), jnp.float32)
        return carry
    lax.fori_loop(0, _N // 16, zero, 0)

    def edge(i, carry):
        o = pl.multiple_of(i * 16, 16)
        plsc.addupdate_scatter(deg_v, [dst_v[pl.ds(o, 16)]], ew_v[pl.ds(o, 16)])
        return carry
    lax.fori_loop(0, _EPW // 16, edge, 0)

    pltpu.sync_copy(deg_v, out_hbm.at[wid])


_deg_kernel = pl.kernel(
    _deg_body,
    out_type=jax.ShapeDtypeStruct((_NW, _N), jnp.float32),
    mesh=plsc.VectorSubcoreMesh(**_MESH),
    compiler_params=_SC_PARAMS,
    scratch_types=[
        pltpu.VMEM((_EPW,), jnp.int32),
        pltpu.VMEM((_EPW,), jnp.float32),
        pltpu.VMEM((_N,), jnp.float32),
    ],
)


def _agg_body(F, table_hbm, src_hbm, dst_hbm, ew_hbm, out_hbm,
              src_v, dst_v, ew_v, rows_v, acc_v, sem):
    nsl = F // 16
    tps = _NS // nsl
    p = _NC * tps
    epp = _E // p
    c = lax.axis_index("c")
    s = lax.axis_index("s")
    fslice = s % nsl
    part = c * tps + s // nsl

    def zero(i, carry):
        for u in range(8):
            acc_v[pl.ds(pl.multiple_of(i * 128 + u * 16, 16), 16)] = (
                jnp.zeros((16,), jnp.float32))
        return carry
    lax.fori_loop(0, (_N * 16) // 128, zero, 0)

    tab = table_hbm.at[fslice]
    lane = lax.iota(jnp.int32, 16)

    def superchunk(sc_i, carry):
        row0 = part * (epp // _CHUNK) + sc_i * _NS
        pltpu.sync_copy(src_hbm.at[pl.ds(row0, _NS)], src_v)
        pltpu.sync_copy(dst_hbm.at[pl.ds(row0, _NS)], dst_v)
        pltpu.sync_copy(ew_hbm.at[pl.ds(row0, _NS)], ew_v)

        def chunk(j, carry2):
            pltpu.async_copy(tab.at[src_v.at[j]], rows_v, sem).wait()

            def group(g, carry3):
                o = pl.multiple_of(g * 16, 16)
                wv = ew_v[j, pl.ds(o, 16)]
                base = dst_v[j, pl.ds(o, 16)] * 16
                row_ids = jnp.full((16,), g * 16, jnp.int32) + lane
                for f in range(16):
                    col = plsc.load_gather(
                        rows_v, [row_ids, jnp.full((16,), f, jnp.int32)])
                    plsc.addupdate_scatter(acc_v, [base + f], col * wv)
                return carry3
            lax.fori_loop(0, _CHUNK // 16, group, 0)
            return carry2
        lax.fori_loop(0, _NS, chunk, 0)
        return carry
    lax.fori_loop(0, epp // (_NS * _CHUNK), superchunk, 0)

    pltpu.sync_copy(acc_v, out_hbm.at[c, s])


def _make_agg(F):
    return pl.kernel(
        functools.partial(_agg_body, F),
        out_type=jax.ShapeDtypeStruct((_NC, _NS, _N * 16), jnp.float32),
        mesh=plsc.VectorSubcoreMesh(**_MESH),
        compiler_params=_SC_PARAMS,
        scratch_types=[
            pltpu.VMEM((_NS, _CHUNK), jnp.int32),
            pltpu.VMEM((_NS, _CHUNK), jnp.int32),
            pltpu.VMEM((_NS, _CHUNK), jnp.float32),
            pltpu.VMEM((_CHUNK, 16), jnp.float32),
            pltpu.VMEM((_N * 16,), jnp.float32),
            pltpu.SemaphoreType.DMA,
        ],
    )


_agg128 = _make_agg(128)
_agg256 = _make_agg(256)


def _agg(agg_fn, xp, F, src2, dst2, ew2):
    nsl = F // 16
    tps = _NS // nsl
    xt = xp.reshape(_N, nsl, 16).transpose(1, 0, 2)
    raw = agg_fn(xt, src2, dst2, ew2)
    acc = raw.reshape(_NC, tps, nsl, _N, 16).transpose(0, 1, 3, 2, 4)
    return acc.reshape(_NC * tps, _N, F)


def _dinv_body(degp_ref, dinv_ref):
    dinv_ref[...] = lax.rsqrt(jnp.sum(degp_ref[...], axis=0) + 1.0)


def _dinv_kernel(degp):
    blk = 512
    return pl.pallas_call(
        _dinv_body,
        grid=(_N // blk,),
        in_specs=[pl.BlockSpec((_NW, blk), lambda i: (0, i))],
        out_specs=pl.BlockSpec((blk,), lambda i: (i,)),
        out_shape=jax.ShapeDtypeStruct((_N,), jnp.float32),
    )(degp)


def _scale_body(x_ref, d_ref, o_ref):
    o_ref[...] = x_ref[...] * d_ref[...]


def _scale_kernel(x, dcol):
    blk = 512
    f = x.shape[1]
    return pl.pallas_call(
        _scale_body,
        grid=(_N // blk,),
        in_specs=[pl.BlockSpec((blk, f), lambda i: (i, 0)),
                  pl.BlockSpec((blk, 1), lambda i: (i, 0))],
        out_specs=pl.BlockSpec((blk, f), lambda i: (i, 0)),
        out_shape=jax.ShapeDtypeStruct((_N, f), jnp.float32),
    )(x, dcol)


def _layer_body(acc_ref, xp_ref, d_ref, W_ref, b_ref, o_ref, *, scale_out):
    d = d_ref[...]
    t = d * (jnp.sum(acc_ref[...], axis=0) + xp_ref[...])
    h = jax.nn.relu(jnp.dot(t, W_ref[...], **_DOT) + b_ref[...])
    o_ref[...] = h * d if scale_out else h


def _layer_kernel(acc, xp, dcol, W, b, scale_out):
    blk = 512
    fi, fo = W.shape
    return pl.pallas_call(
        functools.partial(_layer_body, scale_out=scale_out),
        grid=(_N // blk,),
        in_specs=[pl.BlockSpec((acc.shape[0], blk, fi), lambda i: (0, i, 0)),
                  pl.BlockSpec((blk, fi), lambda i: (i, 0)),
                  pl.BlockSpec((blk, 1), lambda i: (i, 0)),
                  pl.BlockSpec((fi, fo), lambda i: (0, 0)),
                  pl.BlockSpec((1, fo), lambda i: (0, 0))],
        out_specs=pl.BlockSpec((blk, fo), lambda i: (i, 0)),
        out_shape=jax.ShapeDtypeStruct((_N, fo), jnp.float32),
    )(acc, xp, dcol, W, b.reshape(1, fo))


def _head_body(acc_ref, xp_ref, d_ref, W_ref, b_ref, lW_ref, lb_ref, o_ref):
    t = d_ref[...] * (jnp.sum(acc_ref[...], axis=0) + xp_ref[...])
    h = jax.nn.relu(jnp.dot(t, W_ref[...], **_DOT) + b_ref[...])
    o_ref[...] = jnp.dot(h, lW_ref[...], **_DOT) + lb_ref[...]


def _head_kernel(acc, xp, dcol, W, b, lW, lb):
    blk = 512
    fi, fo = W.shape
    fh = lW.shape[1]
    return pl.pallas_call(
        _head_body,
        grid=(_N // blk,),
        in_specs=[pl.BlockSpec((acc.shape[0], blk, fi), lambda i: (0, i, 0)),
                  pl.BlockSpec((blk, fi), lambda i: (i, 0)),
                  pl.BlockSpec((blk, 1), lambda i: (i, 0)),
                  pl.BlockSpec((fi, fo), lambda i: (0, 0)),
                  pl.BlockSpec((1, fo), lambda i: (0, 0)),
                  pl.BlockSpec((fo, fh), lambda i: (0, 0)),
                  pl.BlockSpec((1, fh), lambda i: (0, 0))],
        out_specs=pl.BlockSpec((blk, fh), lambda i: (i, 0)),
        out_shape=jax.ShapeDtypeStruct((_N, fh), jnp.float32),
    )(acc, xp, dcol, W, b.reshape(1, fo), lW, lb.reshape(1, fh))


def _iota_body(o_ref):
    rows = o_ref.shape[1]
    base = pl.program_id(1) * rows
    r = lax.broadcasted_iota(jnp.int32, o_ref.shape, 1) + base
    cidx = lax.broadcasted_iota(jnp.int32, o_ref.shape, 2)
    o_ref[...] = jnp.where(pl.program_id(0) == 0, r, cidx)


def _edge_iota():
    rows = 512
    out = pl.pallas_call(
        _iota_body,
        grid=(2, _N // rows),
        out_specs=pl.BlockSpec((1, rows, _N), lambda i, j: (i, j, 0)),
        out_shape=jax.ShapeDtypeStruct((2, _N, _N), jnp.int32),
    )()
    return out.reshape(2, _N * _N)


def kernel(z_, edge_index, edge_attr, W1, b1, W2, b2, linW, linb):
    new_edge_index = _edge_iota()

    src2 = edge_index[0].reshape(_E // _CHUNK, _CHUNK)
    dst2 = edge_index[1].reshape(_E // _CHUNK, _CHUNK)
    ew2 = edge_attr.reshape(_E // _CHUNK, _CHUNK)
    dstd = edge_index[1].reshape(_NW, _EPW)
    ewd = edge_attr.reshape(_NW, _EPW)

    degp = _deg_kernel(dstd, ewd)
    dinv = _dinv_kernel(degp)
    dcol = dinv.reshape(_N, 1)

    x0p = _scale_kernel(z_, dcol)
    acc1 = _agg(_agg128, x0p, 128, src2, dst2, ew2)
    x1p = _layer_kernel(acc1, x0p, dcol, W1, b1, scale_out=True)
    acc2 = _agg(_agg256, x1p, 256, src2, dst2, ew2)
    out = _head_kernel(acc2, x1p, dcol, W2, b2, linW, linb)
    return (out, new_edge_index)


# speedup vs baseline: 92.2622x; 1.8901x over previous
import functools

import jax
import jax.numpy as jnp
from jax import lax
from jax.experimental import pallas as pl
from jax.experimental.pallas import tpu as pltpu
from jax.experimental.pallas import tpu_sc as plsc

_N = 4096
_E = 65536
_NC = 2
_NS = 16
_NW = _NC * _NS
_EPW = _E // _NW
_CHUNK = 128
_NCHUNK = _EPW // _CHUNK

_MESH = dict(core_axis_name="c", subcore_axis_name="s")
_SC_PARAMS = pltpu.CompilerParams(
    needs_layout_passes=False, use_tc_tiling_on_sc=False)
_DOT = dict(preferred_element_type=jnp.float32, precision=lax.Precision.HIGHEST)


def _deg_body(dst_hbm, ew_hbm, out_hbm, dst_v, ew_v, deg_v):
    c = lax.axis_index("c")
    s = lax.axis_index("s")
    wid = s * _NC + c
    pltpu.sync_copy(dst_hbm.at[wid], dst_v)
    pltpu.sync_copy(ew_hbm.at[wid], ew_v)

    def zero(i, carry):
        deg_v[pl.ds(pl.multiple_of(i * 16, 16), 16)] = jnp.zeros((16,), jnp.float32)
        return carry
    lax.fori_loop(0, _N // 16, zero, 0)

    def edge(i, carry):
        o = pl.multiple_of(i * 16, 16)
        plsc.addupdate_scatter(deg_v, [dst_v[pl.ds(o, 16)]], ew_v[pl.ds(o, 16)])
        return carry
    lax.fori_loop(0, _EPW // 16, edge, 0)

    pltpu.sync_copy(deg_v, out_hbm.at[wid])


_deg_kernel = pl.kernel(
    _deg_body,
    out_type=jax.ShapeDtypeStruct((_NW, _N), jnp.float32),
    mesh=plsc.VectorSubcoreMesh(**_MESH),
    compiler_params=_SC_PARAMS,
    scratch_types=[
        pltpu.VMEM((_EPW,), jnp.int32),
        pltpu.VMEM((_EPW,), jnp.float32),
        pltpu.VMEM((_N,), jnp.float32),
    ],
)


def _agg_body(F, table_hbm, src_hbm, dst_hbm, ew_hbm, out_hbm,
              src_v, dst_v, ew_v, rows_v, acc_v, sem):
    nsl = F // 16
    tps = _NS // nsl
    p = _NC * tps
    epp = _E // p
    c = lax.axis_index("c")
    s = lax.axis_index("s")
    fslice = s % nsl
    part = c * tps + s // nsl

    def zero(i, carry):
        for u in range(8):
            acc_v[pl.ds(pl.multiple_of(i * 128 + u * 16, 16), 16)] = (
                jnp.zeros((16,), jnp.float32))
        return carry
    lax.fori_loop(0, (_N * 16) // 128, zero, 0)

    tab = table_hbm.at[fslice]
    lane = lax.iota(jnp.int32, 16)

    def superchunk(sc_i, carry):
        row0 = part * (epp // _CHUNK) + sc_i * _NS
        pltpu.sync_copy(src_hbm.at[pl.ds(row0, _NS)], src_v)
        pltpu.sync_copy(dst_hbm.at[pl.ds(row0, _NS)], dst_v)
        pltpu.sync_copy(ew_hbm.at[pl.ds(row0, _NS)], ew_v)

        def fire(j, carry2):
            pltpu.async_copy(tab.at[src_v.at[j]], rows_v.at[j], sem)
            return carry2
        lax.fori_loop(0, _NS, fire, 0)

        def drain(j, carry2):
            pltpu.make_async_copy(tab.at[src_v.at[j]], rows_v.at[j], sem).wait()
            return carry2
        lax.fori_loop(0, _NS, drain, 0)

        def chunk(j, carry2):
            def group(g, carry3):
                o = pl.multiple_of(g * 16, 16)
                dstv = dst_v[j, pl.ds(o, 16)]
                wv = ew_v[j, pl.ds(o, 16)]
                for t in range(16):
                    pick = jnp.full((16,), t, jnp.int32)
                    dsts = dstv.at[pick].get(mode="promise_in_bounds")
                    ws = wv.at[pick].get(mode="promise_in_bounds")
                    row = rows_v[j, g * 16 + t, :]
                    plsc.addupdate_scatter(acc_v, [dsts * 16 + lane], row * ws)
                return carry3
            lax.fori_loop(0, _CHUNK // 16, group, 0)
            return carry2
        lax.fori_loop(0, _NS, chunk, 0)
        return carry
    lax.fori_loop(0, epp // (_NS * _CHUNK), superchunk, 0)

    pltpu.sync_copy(acc_v, out_hbm.at[c, s])


def _make_agg(F):
    return pl.kernel(
        functools.partial(_agg_body, F),
        out_type=jax.ShapeDtypeStruct((_NC, _NS, _N * 16), jnp.float32),
        mesh=plsc.VectorSubcoreMesh(**_MESH),
        compiler_params=_SC_PARAMS,
        scratch_types=[
            pltpu.VMEM((_NS, _CHUNK), jnp.int32),
            pltpu.VMEM((_NS, _CHUNK), jnp.int32),
            pltpu.VMEM((_NS, _CHUNK), jnp.float32),
            pltpu.VMEM((_NS, _CHUNK, 16), jnp.float32),
            pltpu.VMEM((_N * 16,), jnp.float32),
            pltpu.SemaphoreType.DMA,
        ],
    )


_agg128 = _make_agg(128)
_agg256 = _make_agg(256)


def _agg(agg_fn, xp, F, src2, dst2, ew2):
    nsl = F // 16
    tps = _NS // nsl
    xt = xp.reshape(_N, nsl, 16).transpose(1, 0, 2)
    raw = agg_fn(xt, src2, dst2, ew2)
    acc = raw.reshape(_NC, tps, nsl, _N, 16).transpose(0, 1, 3, 2, 4)
    return acc.reshape(_NC * tps, _N, F)


def _dinv_body(degp_ref, dinv_ref):
    dinv_ref[...] = lax.rsqrt(jnp.sum(degp_ref[...], axis=0) + 1.0)


def _dinv_kernel(degp):
    blk = 512
    return pl.pallas_call(
        _dinv_body,
        grid=(_N // blk,),
        in_specs=[pl.BlockSpec((_NW, blk), lambda i: (0, i))],
        out_specs=pl.BlockSpec((blk,), lambda i: (i,)),
        out_shape=jax.ShapeDtypeStruct((_N,), jnp.float32),
    )(degp)


def _scale_body(x_ref, d_ref, o_ref):
    o_ref[...] = x_ref[...] * d_ref[...]


def _scale_kernel(x, dcol):
    blk = 512
    f = x.shape[1]
    return pl.pallas_call(
        _scale_body,
        grid=(_N // blk,),
        in_specs=[pl.BlockSpec((blk, f), lambda i: (i, 0)),
                  pl.BlockSpec((blk, 1), lambda i: (i, 0))],
        out_specs=pl.BlockSpec((blk, f), lambda i: (i, 0)),
        out_shape=jax.ShapeDtypeStruct((_N, f), jnp.float32),
    )(x, dcol)


def _layer_body(acc_ref, xp_ref, d_ref, W_ref, b_ref, o_ref, *, scale_out):
    d = d_ref[...]
    t = d * (jnp.sum(acc_ref[...], axis=0) + xp_ref[...])
    h = jax.nn.relu(jnp.dot(t, W_ref[...], **_DOT) + b_ref[...])
    o_ref[...] = h * d if scale_out else h


def _layer_kernel(acc, xp, dcol, W, b, scale_out):
    blk = 512
    fi, fo = W.shape
    return pl.pallas_call(
        functools.partial(_layer_body, scale_out=scale_out),
        grid=(_N // blk,),
        in_specs=[pl.BlockSpec((acc.shape[0], blk, fi), lambda i: (0, i, 0)),
                  pl.BlockSpec((blk, fi), lambda i: (i, 0)),
                  pl.BlockSpec((blk, 1), lambda i: (i, 0)),
                  pl.BlockSpec((fi, fo), lambda i: (0, 0)),
                  pl.BlockSpec((1, fo), lambda i: (0, 0))],
        out_specs=pl.BlockSpec((blk, fo), lambda i: (i, 0)),
        out_shape=jax.ShapeDtypeStruct((_N, fo), jnp.float32),
    )(acc, xp, dcol, W, b.reshape(1, fo))


def _head_body(acc_ref, xp_ref, d_ref, W_ref, b_ref, lW_ref, lb_ref, o_ref):
    t = d_ref[...] * (jnp.sum(acc_ref[...], axis=0) + xp_ref[...])
    h = jax.nn.relu(jnp.dot(t, W_ref[...], **_DOT) + b_ref[...])
    o_ref[...] = jnp.dot(h, lW_ref[...], **_DOT) + lb_ref[...]


def _head_kernel(acc, xp, dcol, W, b, lW, lb):
    blk = 512
    fi, fo = W.shape
    fh = lW.shape[1]
    return pl.pallas_call(
        _head_body,
        grid=(_N // blk,),
        in_specs=[pl.BlockSpec((acc.shape[0], blk, fi), lambda i: (0, i, 0)),
                  pl.BlockSpec((blk, fi), lambda i: (i, 0)),
                  pl.BlockSpec((blk, 1), lambda i: (i, 0)),
                  pl.BlockSpec((fi, fo), lambda i: (0, 0)),
                  pl.BlockSpec((1, fo), lambda i: (0, 0)),
                  pl.BlockSpec((fo, fh), lambda i: (0, 0)),
                  pl.BlockSpec((1, fh), lambda i: (0, 0))],
        out_specs=pl.BlockSpec((blk, fh), lambda i: (i, 0)),
        out_shape=jax.ShapeDtypeStruct((_N, fh), jnp.float32),
    )(acc, xp, dcol, W, b.reshape(1, fo), lW, lb.reshape(1, fh))


def _iota_body(o_ref):
    blk = o_ref.shape[1]
    k = lax.broadcasted_iota(jnp.int32, o_ref.shape, 1) + pl.program_id(0) * blk
    row = lax.broadcasted_iota(jnp.int32, o_ref.shape, 0)
    o_ref[...] = jnp.where(row == 0, k >> 12, k & (_N - 1))


def _edge_iota():
    blk = 1 << 20
    total = _N * _N
    return pl.pallas_call(
        _iota_body,
        grid=(total // blk,),
        out_specs=pl.BlockSpec((2, blk), lambda i: (0, i)),
        out_shape=jax.ShapeDtypeStruct((2, total), jnp.int32),
    )()


def kernel(z_, edge_index, edge_attr, W1, b1, W2, b2, linW, linb):
    new_edge_index = _edge_iota()

    src2 = edge_index[0].reshape(_E // _CHUNK, _CHUNK)
    dst2 = edge_index[1].reshape(_E // _CHUNK, _CHUNK)
    ew2 = edge_attr.reshape(_E // _CHUNK, _CHUNK)
    dstd = edge_index[1].reshape(_NW, _EPW)
    ewd = edge_attr.reshape(_NW, _EPW)

    degp = _deg_kernel(dstd, ewd)
    dinv = _dinv_kernel(degp)
    dcol = dinv.reshape(_N, 1)

    x0p = _scale_kernel(z_, dcol)
    acc1 = _agg(_agg128, x0p, 128, src2, dst2, ew2)
    x1p = _layer_kernel(acc1, x0p, dcol, W1, b1, scale_out=True)
    acc2 = _agg(_agg256, x1p, 256, src2, dst2, ew2)
    out = _head_kernel(acc2, x1p, dcol, W2, b2, linW, linb)
    return (out, new_edge_index)


# speedup vs baseline: 117.4689x; 1.2732x over previous
import functools

import jax
import jax.numpy as jnp
from jax import lax
from jax.experimental import pallas as pl
from jax.experimental.pallas import tpu as pltpu
from jax.experimental.pallas import tpu_sc as plsc

_N = 4096
_E = 65536
_NC = 2
_NS = 16
_NW = _NC * _NS
_EPW = _E // _NW
_CHUNK = 128
_NCHUNK = _EPW // _CHUNK

_MESH = dict(core_axis_name="c", subcore_axis_name="s")
_SC_PARAMS = pltpu.CompilerParams(
    needs_layout_passes=False, use_tc_tiling_on_sc=False)
_DOT = dict(preferred_element_type=jnp.float32, precision=lax.Precision.HIGHEST)


def _deg_body(dst_hbm, ew_hbm, out_hbm, dst_v, ew_v, deg_v):
    c = lax.axis_index("c")
    s = lax.axis_index("s")
    wid = s * _NC + c
    pltpu.sync_copy(dst_hbm.at[wid], dst_v)
    pltpu.sync_copy(ew_hbm.at[wid], ew_v)

    def zero(i, carry):
        deg_v[pl.ds(pl.multiple_of(i * 16, 16), 16)] = jnp.zeros((16,), jnp.float32)
        return carry
    lax.fori_loop(0, _N // 16, zero, 0)

    def edge(i, carry):
        o = pl.multiple_of(i * 16, 16)
        plsc.addupdate_scatter(deg_v, [dst_v[pl.ds(o, 16)]], ew_v[pl.ds(o, 16)])
        return carry
    lax.fori_loop(0, _EPW // 16, edge, 0)

    pltpu.sync_copy(deg_v, out_hbm.at[wid])


_deg_kernel = pl.kernel(
    _deg_body,
    out_type=jax.ShapeDtypeStruct((_NW, _N), jnp.float32),
    mesh=plsc.VectorSubcoreMesh(**_MESH),
    compiler_params=_SC_PARAMS,
    scratch_types=[
        pltpu.VMEM((_EPW,), jnp.int32),
        pltpu.VMEM((_EPW,), jnp.float32),
        pltpu.VMEM((_N,), jnp.float32),
    ],
)


def _agg_body(F, table_hbm, src_hbm, dst_hbm, ew_hbm, out_hbm,
              src_v, dst_v, ew_v, rows_v, acc_v, sem):
    nsl = F // 16
    tps = _NS // nsl
    p = _NC * tps
    epp = _E // p
    c = lax.axis_index("c")
    s = lax.axis_index("s")
    fslice = s % nsl
    part = c * tps + s // nsl

    def zero(i, carry):
        for u in range(8):
            acc_v[pl.ds(pl.multiple_of(i * 128 + u * 16, 16), 16)] = (
                jnp.zeros((16,), jnp.float32))
        return carry
    lax.fori_loop(0, (_N * 16) // 128, zero, 0)

    tab = table_hbm.at[fslice]
    lane = lax.iota(jnp.int32, 16)

    def superchunk(sc_i, carry):
        row0 = part * (epp // _CHUNK) + sc_i * _NS
        pltpu.sync_copy(src_hbm.at[pl.ds(row0, _NS)], src_v)
        pltpu.sync_copy(dst_hbm.at[pl.ds(row0, _NS)], dst_v)
        pltpu.sync_copy(ew_hbm.at[pl.ds(row0, _NS)], ew_v)

        def fire(j, carry2):
            pltpu.async_copy(tab.at[src_v.at[j]], rows_v.at[j], sem)
            return carry2
        lax.fori_loop(0, _NS, fire, 0)

        def drain(j, carry2):
            pltpu.make_async_copy(tab.at[src_v.at[j]], rows_v.at[j], sem).wait()
            return carry2
        lax.fori_loop(0, _NS, drain, 0)

        def chunk(j, carry2):
            @plsc.parallel_loop(0, _CHUNK // 16, unroll=2)
            def group(g):
                o = pl.multiple_of(g * 16, 16)
                dstv = dst_v[j, pl.ds(o, 16)]
                wv = ew_v[j, pl.ds(o, 16)]
                for t in range(16):
                    pick = jnp.full((16,), t, jnp.int32)
                    dsts = dstv.at[pick].get(mode="promise_in_bounds")
                    ws = wv.at[pick].get(mode="promise_in_bounds")
                    row = rows_v[j, g * 16 + t, :]
                    plsc.addupdate_scatter(acc_v, [dsts * 16 + lane], row * ws)
            return carry2
        lax.fori_loop(0, _NS, chunk, 0)
        return carry
    lax.fori_loop(0, epp // (_NS * _CHUNK), superchunk, 0)

    pltpu.sync_copy(acc_v, out_hbm.at[c, s])


def _make_agg(F):
    return pl.kernel(
        functools.partial(_agg_body, F),
        out_type=jax.ShapeDtypeStruct((_NC, _NS, _N * 16), jnp.float32),
        mesh=plsc.VectorSubcoreMesh(**_MESH),
        compiler_params=_SC_PARAMS,
        scratch_types=[
            pltpu.VMEM((_NS, _CHUNK), jnp.int32),
            pltpu.VMEM((_NS, _CHUNK), jnp.int32),
            pltpu.VMEM((_NS, _CHUNK), jnp.float32),
            pltpu.VMEM((_NS, _CHUNK, 16), jnp.float32),
            pltpu.VMEM((_N * 16,), jnp.float32),
            pltpu.SemaphoreType.DMA,
        ],
    )


_agg128 = _make_agg(128)
_agg256 = _make_agg(256)


def _agg(agg_fn, xp, F, src2, dst2, ew2):
    nsl = F // 16
    tps = _NS // nsl
    xt = xp.reshape(_N, nsl, 16).transpose(1, 0, 2)
    raw = agg_fn(xt, src2, dst2, ew2)
    acc = raw.reshape(_NC, tps, nsl, _N, 16).transpose(0, 1, 3, 2, 4)
    return acc.reshape(_NC * tps, _N, F)


def _dinv_body(degp_ref, dinv_ref):
    dinv_ref[...] = lax.rsqrt(jnp.sum(degp_ref[...], axis=0) + 1.0)


def _dinv_kernel(degp):
    blk = 512
    return pl.pallas_call(
        _dinv_body,
        grid=(_N // blk,),
        in_specs=[pl.BlockSpec((_NW, blk), lambda i: (0, i))],
        out_specs=pl.BlockSpec((blk,), lambda i: (i,)),
        out_shape=jax.ShapeDtypeStruct((_N,), jnp.float32),
    )(degp)


def _scale_body(x_ref, d_ref, o_ref):
    o_ref[...] = x_ref[...] * d_ref[...]


def _scale_kernel(x, dcol):
    blk = 512
    f = x.shape[1]
    return pl.pallas_call(
        _scale_body,
        grid=(_N // blk,),
        in_specs=[pl.BlockSpec((blk, f), lambda i: (i, 0)),
                  pl.BlockSpec((blk, 1), lambda i: (i, 0))],
        out_specs=pl.BlockSpec((blk, f), lambda i: (i, 0)),
        out_shape=jax.ShapeDtypeStruct((_N, f), jnp.float32),
    )(x, dcol)


def _layer_body(acc_ref, xp_ref, d_ref, W_ref, b_ref, o_ref, *, scale_out):
    d = d_ref[...]
    t = d * (jnp.sum(acc_ref[...], axis=0) + xp_ref[...])
    h = jax.nn.relu(jnp.dot(t, W_ref[...], **_DOT) + b_ref[...])
    o_ref[...] = h * d if scale_out else h


def _layer_kernel(acc, xp, dcol, W, b, scale_out):
    blk = 512
    fi, fo = W.shape
    return pl.pallas_call(
        functools.partial(_layer_body, scale_out=scale_out),
        grid=(_N // blk,),
        in_specs=[pl.BlockSpec((acc.shape[0], blk, fi), lambda i: (0, i, 0)),
                  pl.BlockSpec((blk, fi), lambda i: (i, 0)),
                  pl.BlockSpec((blk, 1), lambda i: (i, 0)),
                  pl.BlockSpec((fi, fo), lambda i: (0, 0)),
                  pl.BlockSpec((1, fo), lambda i: (0, 0))],
        out_specs=pl.BlockSpec((blk, fo), lambda i: (i, 0)),
        out_shape=jax.ShapeDtypeStruct((_N, fo), jnp.float32),
    )(acc, xp, dcol, W, b.reshape(1, fo))


def _head_body(acc_ref, xp_ref, d_ref, W_ref, b_ref, lW_ref, lb_ref, o_ref):
    t = d_ref[...] * (jnp.sum(acc_ref[...], axis=0) + xp_ref[...])
    h = jax.nn.relu(jnp.dot(t, W_ref[...], **_DOT) + b_ref[...])
    o_ref[...] = jnp.dot(h, lW_ref[...], **_DOT) + lb_ref[...]


def _head_kernel(acc, xp, dcol, W, b, lW, lb):
    blk = 512
    fi, fo = W.shape
    fh = lW.shape[1]
    return pl.pallas_call(
        _head_body,
        grid=(_N // blk,),
        in_specs=[pl.BlockSpec((acc.shape[0], blk, fi), lambda i: (0, i, 0)),
                  pl.BlockSpec((blk, fi), lambda i: (i, 0)),
                  pl.BlockSpec((blk, 1), lambda i: (i, 0)),
                  pl.BlockSpec((fi, fo), lambda i: (0, 0)),
                  pl.BlockSpec((1, fo), lambda i: (0, 0)),
                  pl.BlockSpec((fo, fh), lambda i: (0, 0)),
                  pl.BlockSpec((1, fh), lambda i: (0, 0))],
        out_specs=pl.BlockSpec((blk, fh), lambda i: (i, 0)),
        out_shape=jax.ShapeDtypeStruct((_N, fh), jnp.float32),
    )(acc, xp, dcol, W, b.reshape(1, fo), lW, lb.reshape(1, fh))


def _iota_body(o_ref):
    blk = o_ref.shape[1]
    k = lax.broadcasted_iota(jnp.int32, o_ref.shape, 1) + pl.program_id(0) * blk
    row = lax.broadcasted_iota(jnp.int32, o_ref.shape, 0)
    o_ref[...] = jnp.where(row == 0, k >> 12, k & (_N - 1))


def _edge_iota():
    blk = 1 << 20
    total = _N * _N
    return pl.pallas_call(
        _iota_body,
        grid=(total // blk,),
        out_specs=pl.BlockSpec((2, blk), lambda i: (0, i)),
        out_shape=jax.ShapeDtypeStruct((2, total), jnp.int32),
    )()


def kernel(z_, edge_index, edge_attr, W1, b1, W2, b2, linW, linb):
    new_edge_index = _edge_iota()

    src2 = edge_index[0].reshape(_E // _CHUNK, _CHUNK)
    dst2 = edge_index[1].reshape(_E // _CHUNK, _CHUNK)
    ew2 = edge_attr.reshape(_E // _CHUNK, _CHUNK)
    dstd = edge_index[1].reshape(_NW, _EPW)
    ewd = edge_attr.reshape(_NW, _EPW)

    degp = _deg_kernel(dstd, ewd)
    dinv = _dinv_kernel(degp)
    dcol = dinv.reshape(_N, 1)

    x0p = _scale_kernel(z_, dcol)
    acc1 = _agg(_agg128, x0p, 128, src2, dst2, ew2)
    x1p = _layer_kernel(acc1, x0p, dcol, W1, b1, scale_out=True)
    acc2 = _agg(_agg256, x1p, 256, src2, dst2, ew2)
    out = _head_kernel(acc2, x1p, dcol, W2, b2, linW, linb)
    return (out, new_edge_index)
